# Initial kernel scaffold; baseline (speedup 1.0000x reference)
import jax
import jax.numpy as jnp
from jax.experimental import pallas as pl


def kernel(x, edge_index, W1l, b1l, W1r, Wlin1, blin1, W2l, b2l, W2r, Wlin2, blin2):
    raise NotImplementedError("write your pallas kernel here")


# speedup vs baseline: 6.9038x; 6.9038x over previous
import functools

import jax
import jax.numpy as jnp
from jax import lax
from jax.experimental import pallas as pl
from jax.experimental.pallas import tpu as pltpu
from jax.experimental.pallas import tpu_sc as plsc

N = 10000
E = 320000
D = 128

NC = 2
NS = 16
NW = NC * NS
CH = 128
NCHUNK = E // CH
NP = 10240
ROWS_PER_TILE = NP // NS
FULL_ITERS = NCHUNK // NW


def _sc_agg_deg_body(x_hbm, src_hbm, dst_hbm, zc_hbm, acc_out, deg_out,
                     sidx_v, didx_v, rows_v, sem, acc_sh, degv):
    c = lax.axis_index("c")
    s = lax.axis_index("s")
    w = s * NC + c
    rbase = s * ROWS_PER_TILE
    zeros16 = jnp.zeros((16,), jnp.float32)
    ones16 = jnp.ones((16,), jnp.float32)

    pltpu.sync_copy(zc_hbm, acc_sh.at[pl.ds(rbase, ROWS_PER_TILE)])

    def zbody(i, carry):
        for k in range(16):
            degv[pl.ds((i * 16 + k) * 16, 16)] = zeros16
        return carry

    lax.fori_loop(0, NP // 256, zbody, 0)
    plsc.subcore_barrier()

    n_iter = FULL_ITERS + jnp.where(w < NCHUNK - FULL_ITERS * NW, 1, 0)

    def body(i, carry):
        eb = (w + i * NW) * CH
        pltpu.sync_copy(src_hbm.at[pl.ds(eb, CH)], sidx_v)
        pltpu.sync_copy(dst_hbm.at[pl.ds(eb, CH)], didx_v)
        pltpu.async_copy(x_hbm.at[sidx_v], rows_v, sem).wait()
        pltpu.sync_copy(rows_v, acc_sh.at[didx_v], add=True)
        for j in range(CH // 16):
            plsc.addupdate_scatter(degv, [didx_v[pl.ds(j * 16, 16)]], ones16)
        return carry

    lax.fori_loop(0, n_iter, body, 0)
    plsc.subcore_barrier()

    pltpu.sync_copy(acc_sh.at[pl.ds(rbase, ROWS_PER_TILE)],
                    acc_out.at[c, pl.ds(rbase, ROWS_PER_TILE)])
    pltpu.sync_copy(degv, deg_out.at[w])


def _sc_agg_body(x_hbm, src_hbm, dst_hbm, zc_hbm, acc_out,
                 sidx_v, didx_v, rows_v, sem, acc_sh):
    c = lax.axis_index("c")
    s = lax.axis_index("s")
    w = s * NC + c
    rbase = s * ROWS_PER_TILE

    pltpu.sync_copy(zc_hbm, acc_sh.at[pl.ds(rbase, ROWS_PER_TILE)])
    plsc.subcore_barrier()

    n_iter = FULL_ITERS + jnp.where(w < NCHUNK - FULL_ITERS * NW, 1, 0)

    def body(i, carry):
        eb = (w + i * NW) * CH
        pltpu.sync_copy(src_hbm.at[pl.ds(eb, CH)], sidx_v)
        pltpu.sync_copy(dst_hbm.at[pl.ds(eb, CH)], didx_v)
        pltpu.async_copy(x_hbm.at[sidx_v], rows_v, sem).wait()
        pltpu.sync_copy(rows_v, acc_sh.at[didx_v], add=True)
        return carry

    lax.fori_loop(0, n_iter, body, 0)
    plsc.subcore_barrier()

    pltpu.sync_copy(acc_sh.at[pl.ds(rbase, ROWS_PER_TILE)],
                    acc_out.at[c, pl.ds(rbase, ROWS_PER_TILE)])


def _sc_aggregate(x, src, dst, with_deg):
    mesh = plsc.VectorSubcoreMesh(core_axis_name="c", subcore_axis_name="s")
    zc = jnp.zeros((ROWS_PER_TILE, D), jnp.float32)
    scratch = [
        pltpu.VMEM((CH,), jnp.int32),
        pltpu.VMEM((CH,), jnp.int32),
        pltpu.VMEM((CH, D), jnp.float32),
        pltpu.SemaphoreType.DMA,
        pltpu.VMEM_SHARED((NP, D), jnp.float32),
    ]
    if with_deg:
        scratch.append(pltpu.VMEM((NP,), jnp.float32))
        kern = pl.kernel(
            _sc_agg_deg_body,
            out_type=(jax.ShapeDtypeStruct((NC, NP, D), jnp.float32),
                      jax.ShapeDtypeStruct((NW, NP), jnp.float32)),
            mesh=mesh,
            scratch_types=scratch,
            compiler_params=pltpu.CompilerParams(needs_layout_passes=False),
        )
        return kern(x, src, dst, zc)
    kern = pl.kernel(
        _sc_agg_body,
        out_type=jax.ShapeDtypeStruct((NC, NP, D), jnp.float32),
        mesh=mesh,
        scratch_types=scratch,
    )
    return kern(x, src, dst, zc)


def _tc_layer_body(relu, acc_ref, deg_ref, x_ref, wl_ref, wc_ref, b_ref, o_ref):
    a = acc_ref[0] + acc_ref[1]
    d = jnp.sum(deg_ref[...], axis=0)
    dclip = jnp.maximum(d, 1.0)[:, None]
    mean = a / dclip
    y = (jnp.dot(mean, wl_ref[...], preferred_element_type=jnp.float32)
         + jnp.dot(x_ref[...], wc_ref[...], preferred_element_type=jnp.float32)
         + b_ref[...])
    if relu:
        y = jnp.maximum(y, 0.0)
    o_ref[...] = y


def _tc_layer(acc, deg, x, wl, wc, b, relu):
    R = 2048
    grid = (NP // R,)
    return pl.pallas_call(
        functools.partial(_tc_layer_body, relu),
        grid=grid,
        in_specs=[
            pl.BlockSpec((NC, R, D), lambda i: (0, i, 0)),
            pl.BlockSpec((NW, R), lambda i: (0, i)),
            pl.BlockSpec((R, D), lambda i: (i, 0)),
            pl.BlockSpec((D, D), lambda i: (0, 0)),
            pl.BlockSpec((D, D), lambda i: (0, 0)),
            pl.BlockSpec((1, D), lambda i: (0, 0)),
        ],
        out_specs=pl.BlockSpec((R, D), lambda i: (i, 0)),
        out_shape=jax.ShapeDtypeStruct((NP, D), jnp.float32),
    )(acc, deg, x, wl, wc, b)


def kernel(x, edge_index, W1l, b1l, W1r, Wlin1, blin1, W2l, b2l, W2r, Wlin2, blin2):
    src = edge_index[0]
    dst = edge_index[1]
    xp = jnp.concatenate([x, jnp.zeros((NP - N, D), jnp.float32)], axis=0)
    acc1, deg = _sc_aggregate(xp, src, dst, with_deg=True)
    h = _tc_layer(acc1, deg, xp, W1l, W1r + Wlin1,
                  (b1l + blin1).reshape(1, D), relu=True)
    acc2 = _sc_aggregate(h, src, dst, with_deg=False)
    out = _tc_layer(acc2, deg, h, W2l, W2r + Wlin2,
                    (b2l + blin2).reshape(1, D), relu=False)
    return out[:N]
